# baseline (speedup 1.0000x reference)
import functools

import jax
import jax.numpy as jnp
from jax import lax
from jax.experimental import pallas as pl
from jax.experimental.pallas import tpu as pltpu
from jax.experimental.pallas import tpu_sc as plsc

NUM_EMB = 1000000
D = 32
BATCH = 16384
HIST = 200
B = BATCH * HIST

NC = 2
NS = 16
NW = NC * NS

G = 128
NBT = 4
CH = NBT * G
ROWS = B // G
NG = ROWS // NW // NBT
TSZ = NBT * D * G


@functools.partial(
    pl.kernel,
    mesh=plsc.VectorSubcoreMesh(core_axis_name="c", subcore_axis_name="s"),
    compiler_params=pltpu.CompilerParams(use_tc_tiling_on_sc=False,
                                         needs_layout_passes=False),
    out_type=jax.ShapeDtypeStruct((B * D,), jnp.float32),
    scratch_types=[
        pltpu.VMEM((NBT, G), jnp.int32),
        pltpu.VMEM((NBT, G), jnp.int32),
        pltpu.VMEM((CH, D), jnp.float32),
        pltpu.VMEM((CH, D), jnp.float32),
        pltpu.VMEM((TSZ,), jnp.float32),
        pltpu.VMEM((TSZ,), jnp.float32),
        pltpu.SemaphoreType.DMA,
        pltpu.SemaphoreType.DMA,
        pltpu.SemaphoreType.DMA,
        pltpu.SemaphoreType.DMA,
        pltpu.SemaphoreType.DMA,
        pltpu.SemaphoreType.DMA,
    ],
)
def _emb_gather(idx_hbm, table_hbm, out_hbm, idx0, idx1, rows0, rows1,
                t0, t1, isem0, isem1, gsem0, gsem1, ssem0, ssem1):
    wid = lax.axis_index("s") * NC + lax.axis_index("c")
    row0 = wid * (NBT * NG)
    idx_v = (idx0, idx1)
    rows_v = (rows0, rows1)
    t_v = (t0, t1)
    isem = (isem0, isem1)
    gsem = (gsem0, gsem1)
    ssem = (ssem0, ssem1)
    iota16 = lax.broadcasted_iota(jnp.int32, (16,), 0)

    def idx_copy(g, slot):
        return pltpu.make_async_copy(
            idx_hbm.at[pl.ds(row0 + g * NBT, NBT)], idx_v[slot],
            isem[slot])

    def gather_copies(slot):
        return [
            pltpu.make_async_copy(
                table_hbm.at[idx_v[slot].at[u]],
                rows_v[slot].at[pl.ds(u * G, G)],
                gsem[slot])
            for u in range(NBT)
        ]

    def store_copies(g, slot):
        r = row0 + g * NBT
        h = r >> 7
        bt0 = r & 127
        return [
            pltpu.make_async_copy(
                t_v[slot].at[pl.ds(dt * (NBT * 8 * G), NBT * 8 * G)],
                out_hbm.at[pl.ds(h * (4 * 8 * G * G) + dt * (8 * G * G)
                                 + bt0 * (8 * G), NBT * 8 * G)],
                ssem[slot])
            for dt in range(4)
        ]

    idx_copy(0, 0).start()
    idx_copy(0, 0).wait()
    for c in gather_copies(0):
        c.start()
    idx_copy(1, 1).start()

    def pair_body(p, _):
        for s in (0, 1):
            g = 2 * p + s
            o = 1 - s
            rows_s = rows_v[s]
            t_s = t_v[s]

            @pl.when(g + 1 < NG)
            def _():
                idx_copy(g + 1, o).wait()
                for c in gather_copies(o):
                    c.start()

            for c in gather_copies(s):
                c.wait()

            @pl.when(g + 2 < NG)
            def _():
                idx_copy(g + 2, s).start()

            @pl.when(g >= 2)
            def _():
                for c in store_copies(g - 2, s):
                    c.wait()

            def tbody(ds, _, rows_s=rows_s, t_s=t_s):
                for dt in range(4):
                    d_vec = jnp.full((16,), dt * 8, jnp.int32) + ds
                    for u in range(NBT):
                        for bl0 in range(0, G, 16):
                            row_idx = iota16 + (u * G + bl0)
                            v = plsc.load_gather(rows_s, [row_idx, d_vec])
                            t_s[pl.ds((dt * NBT + u) * (8 * G)
                                      + ds * G + bl0, 16)] = v
                return _

            lax.fori_loop(0, 8, tbody, None)

            for c in store_copies(g, s):
                c.start()
        return _

    lax.fori_loop(0, NG // 2, pair_body, None)
    for c in store_copies(0, 0):
        c.wait()
    for c in store_copies(0, 1):
        c.wait()


def kernel(x, table):
    idx2d = x.T.astype(jnp.int32).reshape(ROWS, G)
    flat = _emb_gather(idx2d, table)
    return (flat.reshape(HIST, 4, G, 8, G)
            .transpose(2, 4, 0, 1, 3)
            .reshape(BATCH, HIST, D))


# speedup vs baseline: 1.9931x; 1.9931x over previous
import functools

import jax
import jax.numpy as jnp
from jax import lax
from jax.experimental import pallas as pl
from jax.experimental.pallas import tpu as pltpu
from jax.experimental.pallas import tpu_sc as plsc

NUM_EMB = 1000000
D = 32
BATCH = 16384
HIST = 200
B = BATCH * HIST

NC = 2
NS = 16
NW = NC * NS

G = 128
NBT = 4
CH = NBT * G
ROWS = B // G
NG = ROWS // NW // NBT
TSZ = NBT * D * G


@functools.partial(
    pl.kernel,
    mesh=plsc.VectorSubcoreMesh(core_axis_name="c", subcore_axis_name="s"),
    compiler_params=pltpu.CompilerParams(use_tc_tiling_on_sc=False,
                                         needs_layout_passes=False),
    out_type=jax.ShapeDtypeStruct((B * D,), jnp.float32),
    scratch_types=[
        pltpu.VMEM((NBT, G), jnp.int32),
        pltpu.VMEM((NBT, G), jnp.int32),
        pltpu.VMEM((CH, D), jnp.float32),
        pltpu.VMEM((CH, D), jnp.float32),
        pltpu.VMEM((TSZ,), jnp.float32),
        pltpu.VMEM((TSZ,), jnp.float32),
        pltpu.SemaphoreType.DMA,
        pltpu.SemaphoreType.DMA,
        pltpu.SemaphoreType.DMA,
        pltpu.SemaphoreType.DMA,
        pltpu.SemaphoreType.DMA,
        pltpu.SemaphoreType.DMA,
    ],
)
def _emb_gather(idx_hbm, table_hbm, out_hbm, idx0, idx1, rows0, rows1,
                t0, t1, isem0, isem1, gsem0, gsem1, ssem0, ssem1):
    wid = lax.axis_index("s") * NC + lax.axis_index("c")
    row0 = wid * (NBT * NG)
    idx_v = (idx0, idx1)
    rows_v = (rows0, rows1)
    t_v = (t0, t1)
    isem = (isem0, isem1)
    gsem = (gsem0, gsem1)
    ssem = (ssem0, ssem1)
    iota16 = lax.broadcasted_iota(jnp.int32, (16,), 0)

    def idx_copy(g, slot):
        return pltpu.make_async_copy(
            idx_hbm.at[pl.ds(row0 + g * NBT, NBT)], idx_v[slot],
            isem[slot])

    def gather_copies(slot):
        return [
            pltpu.make_async_copy(
                table_hbm.at[idx_v[slot].at[u]],
                rows_v[slot].at[pl.ds(u * G, G)],
                gsem[slot])
            for u in range(NBT)
        ]

    def store_copies(g, slot):
        r = row0 + g * NBT
        h = r >> 7
        bt0 = r & 127
        return [
            pltpu.make_async_copy(
                t_v[slot].at[pl.ds(dt * (NBT * 8 * G), NBT * 8 * G)],
                out_hbm.at[pl.ds(h * (4 * 8 * G * G) + dt * (8 * G * G)
                                 + bt0 * (8 * G), NBT * 8 * G)],
                ssem[slot])
            for dt in range(4)
        ]

    idx_copy(0, 0).start()
    idx_copy(0, 0).wait()
    for c in gather_copies(0):
        c.start()
    idx_copy(1, 1).start()

    def pair_body(p, _):
        for s in (0, 1):
            g = 2 * p + s
            o = 1 - s
            rows_s = rows_v[s]
            t_s = t_v[s]

            @pl.when(g + 1 < NG)
            def _():
                idx_copy(g + 1, o).wait()
                for c in gather_copies(o):
                    c.start()

            for c in gather_copies(s):
                c.wait()

            @pl.when(g + 2 < NG)
            def _():
                idx_copy(g + 2, s).start()

            @pl.when(g >= 2)
            def _():
                for c in store_copies(g - 2, s):
                    c.wait()

            def tbody(k, _, rows_s=rows_s, t_s=t_s):
                m = (iota16 + k) & 15
                pv = (m >> 3) * (NBT * 8 * G) + (m & 7) * G + iota16
                for d0 in (0, 16):
                    for u in range(NBT):
                        for bl0 in range(0, G, 16):
                            row_idx = iota16 + (u * G + bl0)
                            col_idx = m + d0
                            v = plsc.load_gather(rows_s, [row_idx, col_idx])
                            pos = pv + ((d0 >> 3) * (NBT * 8 * G)
                                        + u * (8 * G) + bl0)
                            plsc.store_scatter(t_s, [pos], v)
                return _

            lax.fori_loop(0, 16, tbody, None)
            plsc.subcore_barrier()

            for c in store_copies(g, s):
                c.start()
        return _

    lax.fori_loop(0, NG // 2, pair_body, None)
    for c in store_copies(0, 0):
        c.wait()
    for c in store_copies(0, 1):
        c.wait()


def kernel(x, table):
    idx2d = x.T.astype(jnp.int32).reshape(ROWS, G)
    flat = _emb_gather(idx2d, table)
    return (flat.reshape(HIST, 4, G, 8, G)
            .transpose(2, 4, 0, 1, 3)
            .reshape(BATCH, HIST, D))


# speedup vs baseline: 2.0282x; 1.0176x over previous
import functools

import jax
import jax.numpy as jnp
from jax import lax
from jax.experimental import pallas as pl
from jax.experimental.pallas import tpu as pltpu
from jax.experimental.pallas import tpu_sc as plsc

NUM_EMB = 1000000
D = 32
BATCH = 16384
HIST = 200
B = BATCH * HIST

NC = 2
NS = 16
NW = NC * NS

G = 128
NBT = 4
CH = NBT * G
ROWS = B // G
NG = ROWS // NW // NBT
TSZ = NBT * D * G


@functools.partial(
    pl.kernel,
    mesh=plsc.VectorSubcoreMesh(core_axis_name="c", subcore_axis_name="s"),
    compiler_params=pltpu.CompilerParams(use_tc_tiling_on_sc=False,
                                         needs_layout_passes=False),
    out_type=jax.ShapeDtypeStruct((B * D,), jnp.float32),
    scratch_types=[
        pltpu.VMEM((NBT, G), jnp.int32),
        pltpu.VMEM((NBT, G), jnp.int32),
        pltpu.VMEM((CH, D), jnp.float32),
        pltpu.VMEM((CH, D), jnp.float32),
        pltpu.VMEM((TSZ,), jnp.float32),
        pltpu.VMEM((TSZ,), jnp.float32),
        pltpu.SemaphoreType.DMA,
        pltpu.SemaphoreType.DMA,
        pltpu.SemaphoreType.DMA,
        pltpu.SemaphoreType.DMA,
        pltpu.SemaphoreType.DMA,
        pltpu.SemaphoreType.DMA,
    ],
)
def _emb_gather(idx_hbm, table_hbm, out_hbm, idx0, idx1, rows0, rows1,
                t0, t1, isem0, isem1, gsem0, gsem1, ssem0, ssem1):
    wid = lax.axis_index("s") * NC + lax.axis_index("c")
    row0 = wid * (NBT * NG)
    idx_v = (idx0, idx1)
    rows_v = (rows0, rows1)
    t_v = (t0, t1)
    isem = (isem0, isem1)
    gsem = (gsem0, gsem1)
    ssem = (ssem0, ssem1)
    iota16 = lax.broadcasted_iota(jnp.int32, (16,), 0)

    def idx_copy(g, slot):
        return pltpu.make_async_copy(
            idx_hbm.at[pl.ds(row0 + g * NBT, NBT)], idx_v[slot],
            isem[slot])

    def gather_copies(slot):
        return [
            pltpu.make_async_copy(
                table_hbm.at[idx_v[slot].at[u]],
                rows_v[slot].at[pl.ds(u * G, G)],
                gsem[slot])
            for u in range(NBT)
        ]

    def gather_drain(slot):
        return pltpu.make_async_copy(
            table_hbm.at[pl.ds(0, CH)], rows_v[slot], gsem[slot])

    def store_drain(slot):
        return pltpu.make_async_copy(
            t_v[slot], out_hbm.at[pl.ds(0, TSZ)], ssem[slot])

    def store_copies(g, slot):
        r = row0 + g * NBT
        h = r >> 7
        bt0 = r & 127
        return [
            pltpu.make_async_copy(
                t_v[slot].at[pl.ds(dt * (NBT * 8 * G), NBT * 8 * G)],
                out_hbm.at[pl.ds(h * (4 * 8 * G * G) + dt * (8 * G * G)
                                 + bt0 * (8 * G), NBT * 8 * G)],
                ssem[slot])
            for dt in range(4)
        ]

    idx_copy(0, 0).start()
    idx_copy(0, 0).wait()
    for c in gather_copies(0):
        c.start()
    idx_copy(1, 1).start()

    def pair_body(p, _):
        for s in (0, 1):
            g = 2 * p + s
            o = 1 - s
            rows_s = rows_v[s]
            t_s = t_v[s]

            @pl.when(g + 1 < NG)
            def _():
                idx_copy(g + 1, o).wait()
                for c in gather_copies(o):
                    c.start()

            gather_drain(s).wait()

            @pl.when(g + 2 < NG)
            def _():
                idx_copy(g + 2, s).start()

            @pl.when(g >= 2)
            def _():
                store_drain(s).wait()

            def tbody(k, _, rows_s=rows_s, t_s=t_s):
                m = (iota16 + k) & 15
                pv = (m >> 3) * (NBT * 8 * G) + (m & 7) * G + iota16
                for d0 in (0, 16):
                    for u in range(NBT):
                        for bl0 in range(0, G, 16):
                            row_idx = iota16 + (u * G + bl0)
                            col_idx = m + d0
                            v = plsc.load_gather(rows_s, [row_idx, col_idx])
                            pos = pv + ((d0 >> 3) * (NBT * 8 * G)
                                        + u * (8 * G) + bl0)
                            plsc.store_scatter(t_s, [pos], v)
                return _

            lax.fori_loop(0, 16, tbody, None)
            plsc.subcore_barrier()

            for c in store_copies(g, s):
                c.start()
        return _

    lax.fori_loop(0, NG // 2, pair_body, None)
    store_drain(0).wait()
    store_drain(1).wait()


def kernel(x, table):
    idx2d = x.T.astype(jnp.int32).reshape(ROWS, G)
    flat = _emb_gather(idx2d, table)
    return (flat.reshape(HIST, 4, G, 8, G)
            .transpose(2, 4, 0, 1, 3)
            .reshape(BATCH, HIST, D))
